# Initial kernel scaffold; baseline (speedup 1.0000x reference)
import jax
import jax.numpy as jnp
from jax.experimental import pallas as pl


def kernel(x, edge_index, W1, b1, Wmu, bmu, Wlv, blv):
    raise NotImplementedError("write your pallas kernel here")


# speedup vs baseline: 8.6810x; 8.6810x over previous
import functools

import jax
import jax.numpy as jnp
from jax import lax
from jax.experimental import pallas as pl
from jax.experimental.pallas import tpu as pltpu
from jax.experimental.pallas import tpu_sc as plsc

_LANES = 128
_NC = 2
_NS = 16
_RPT = 640
_G = 8


def _deg_kernel(N2, R32):
    mesh = plsc.VectorSubcoreMesh(core_axis_name="c", subcore_axis_name="s",
                                  num_cores=_NC, num_subcores=_NS)

    @functools.partial(
        pl.kernel,
        mesh=mesh,
        out_type=jax.ShapeDtypeStruct((2 * N2, _LANES), jnp.float32),
        scratch_types=[
            pltpu.VMEM((R32, _LANES), jnp.int32),
            pltpu.VMEM((_LANES, _LANES), jnp.float32),
            pltpu.VMEM_SHARED((N2, _LANES), jnp.float32),
        ],
    )
    def body(dst_hbm, ones_hbm, out_hbm, dst_v, ones_v, acc_sh):
        c = lax.axis_index("c")
        s = lax.axis_index("s")
        wid = s * _NC + c
        pltpu.sync_copy(dst_hbm.at[wid], dst_v)
        pltpu.sync_copy(ones_hbm.at[pl.ds(0, _LANES)], ones_v)
        r0 = s * _RPT
        pltpu.sync_copy(ones_hbm.at[pl.ds(r0, _RPT)], acc_sh.at[pl.ds(r0, _RPT)])
        plsc.subcore_barrier()

        def step(j, carry):
            pltpu.sync_copy(ones_v, acc_sh.at[dst_v.at[j]], add=True)
            return carry

        lax.fori_loop(0, R32, step, 0)
        plsc.subcore_barrier()
        pltpu.sync_copy(acc_sh.at[pl.ds(r0, _RPT)],
                        out_hbm.at[pl.ds(c * N2 + r0, _RPT)])

    return body


def _agg_kernel(N2, R, C):
    mesh = plsc.VectorSubcoreMesh(core_axis_name="c", subcore_axis_name="s",
                                  num_cores=_NC, num_subcores=_NS)

    @functools.partial(
        pl.kernel,
        mesh=mesh,
        out_type=jax.ShapeDtypeStruct((2 * N2, _LANES), jnp.float32),
        scratch_types=[
            pltpu.VMEM((R // 2, 2 * C), jnp.int32),
            pltpu.VMEM((_G, C), jnp.int32),
            pltpu.VMEM((C, _LANES), jnp.float32),
            pltpu.VMEM((C, _LANES), jnp.float32),
            pltpu.VMEM((C, _LANES), jnp.float32),
            pltpu.VMEM_SHARED((N2, _LANES), jnp.float32),
            pltpu.SemaphoreType.DMA,
            pltpu.SemaphoreType.DMA,
            pltpu.SemaphoreType.DMA,
        ],
    )
    def body(y_hbm, src_hbm, dst_hbm, out_hbm, src_v, dst_st, rows0, rows1,
             rows2, acc_sh, sem0, sem1, sem2):
        c = lax.axis_index("c")
        s = lax.axis_index("s")
        pltpu.sync_copy(src_hbm.at[c, s], src_v)
        r0 = s * _RPT
        pltpu.sync_copy(y_hbm.at[pl.ds(c * N2 + r0, _RPT)],
                        acc_sh.at[pl.ds(r0, _RPT)])
        plsc.subcore_barrier()

        rows = (rows0, rows1, rows2)
        sems = (sem0, sem1, sem2)

        def src_idx(j):
            return src_v.at[j // 2, pl.ds((j % 2) * C, C)]

        def group(g, carry):
            pltpu.sync_copy(dst_hbm.at[s, g], dst_st)
            j0 = g * _G
            h = [pltpu.async_copy(y_hbm.at[src_idx(j0 + b)], rows[b], sems[b])
                 for b in range(3)]
            for k in range(_G):
                b = k % 3
                h[b].wait()
                pltpu.sync_copy(rows[b], acc_sh.at[dst_st.at[k]], add=True)
                if k + 3 < _G:
                    h[b] = pltpu.async_copy(y_hbm.at[src_idx(j0 + k + 3)],
                                            rows[b], sems[b])
            return carry

        lax.fori_loop(0, R // _G, group, 0)
        plsc.subcore_barrier()
        pltpu.sync_copy(acc_sh.at[pl.ds(r0, _RPT)],
                        out_hbm.at[pl.ds(c * N2 + r0, _RPT)])

    return body


def _scale_body(d0_ref, d1_ref, x_ref, dinv_ref, y2_ref):
    dv = lax.rsqrt(d0_ref[:, 0:1] + d1_ref[:, 0:1] - 1.0)
    dinv_ref[...] = dv
    y2_ref[...] = x_ref[...] * dv


def _layer1_body(zl_ref, zr_ref, dv_ref, w_ref, b_ref, y_ref):
    dv = dv_ref[...]
    al = zl_ref[...] * dv
    ar = zr_ref[...] * dv
    pre = jnp.dot(al, w_ref[0:128, :], preferred_element_type=jnp.float32)
    pre = pre + jnp.dot(ar, w_ref[128:256, :], preferred_element_type=jnp.float32)
    pre = pre + b_ref[...]
    y_ref[...] = dv * jnp.maximum(pre, 0.0)


def _heads_body(zl_ref, zr_ref, dv_ref, wmu_ref, bmu_ref, wlv_ref, blv_ref,
                mu_ref, lv_ref):
    dv = dv_ref[...]
    al = zl_ref[...] * dv
    ar = zr_ref[...] * dv
    mu_ref[...] = (jnp.dot(al, wmu_ref[0:128, :], preferred_element_type=jnp.float32)
                   + jnp.dot(ar, wmu_ref[128:256, :], preferred_element_type=jnp.float32)
                   + bmu_ref[...])
    lv_ref[...] = (jnp.dot(al, wlv_ref[0:128, :], preferred_element_type=jnp.float32)
                   + jnp.dot(ar, wlv_ref[128:256, :], preferred_element_type=jnp.float32)
                   + blv_ref[...])


def kernel(x, edge_index, W1, b1, Wmu, bmu, Wlv, blv):
    N, D = x.shape
    E = edge_index.shape[1]
    H = W1.shape[1]
    N2 = _NS * _RPT
    nb = N2 // _RPT

    R32 = -(-E // (_NC * _NS * _LANES))
    Epad = _NC * _NS * R32 * _LANES
    C_AGG = 64
    R_AGG = Epad // (_NS * C_AGG)

    src = edge_index[0]
    dst = edge_index[1]
    padn = Epad - E
    src_p = jnp.concatenate([src, jnp.zeros((padn,), src.dtype)])
    dst_p = jnp.concatenate([dst, jnp.full((padn,), N, dst.dtype)])
    srcs = jnp.stack([src_p, src_p + N2]).reshape(_NC, _NS, R_AGG // 2, 2 * C_AGG)
    dsts = dst_p.reshape(_NS, R_AGG // _G, _G, C_AGG)
    dst32 = dst_p.reshape(_NC * _NS, R32, _LANES)
    ones = jnp.ones((N2, _LANES), jnp.float32)

    degpart = _deg_kernel(N2, R32)(dst32, ones)

    dinv, y0 = pl.pallas_call(
        _scale_body,
        grid=(nb, 2),
        in_specs=[
            pl.BlockSpec((_RPT, 128), lambda i, h: (i, 0)),
            pl.BlockSpec((_RPT, 128), lambda i, h: (nb + i, 0)),
            pl.BlockSpec((_RPT, 128), lambda i, h: (i, h)),
        ],
        out_specs=[
            pl.BlockSpec((_RPT, 1), lambda i, h: (i, 0)),
            pl.BlockSpec((_RPT, 128), lambda i, h: (h * nb + i, 0)),
        ],
        out_shape=[
            jax.ShapeDtypeStruct((N, 1), jnp.float32),
            jax.ShapeDtypeStruct((2 * N2, 128), jnp.float32),
        ],
    )(degpart, degpart, x)

    agg = _agg_kernel(N2, R_AGG, C_AGG)
    z0 = agg(y0, srcs, dsts)

    y1 = pl.pallas_call(
        _layer1_body,
        grid=(nb, 2),
        in_specs=[
            pl.BlockSpec((_RPT, 128), lambda i, h: (i, 0)),
            pl.BlockSpec((_RPT, 128), lambda i, h: (nb + i, 0)),
            pl.BlockSpec((_RPT, 1), lambda i, h: (i, 0)),
            pl.BlockSpec((D, 128), lambda i, h: (0, h)),
            pl.BlockSpec((1, 128), lambda i, h: (0, h)),
        ],
        out_specs=pl.BlockSpec((_RPT, 128), lambda i, h: (h * nb + i, 0)),
        out_shape=jax.ShapeDtypeStruct((2 * N2, 128), jnp.float32),
    )(z0, z0, dinv, W1, b1.reshape(1, H))

    z1 = agg(y1, srcs, dsts)

    mu, lv = pl.pallas_call(
        _heads_body,
        grid=(nb, 2),
        in_specs=[
            pl.BlockSpec((_RPT, 128), lambda i, h: (i, 0)),
            pl.BlockSpec((_RPT, 128), lambda i, h: (nb + i, 0)),
            pl.BlockSpec((_RPT, 1), lambda i, h: (i, 0)),
            pl.BlockSpec((H, 128), lambda i, h: (0, h)),
            pl.BlockSpec((1, 128), lambda i, h: (0, h)),
            pl.BlockSpec((H, 128), lambda i, h: (0, h)),
            pl.BlockSpec((1, 128), lambda i, h: (0, h)),
        ],
        out_specs=[
            pl.BlockSpec((_RPT, 128), lambda i, h: (i, h)),
            pl.BlockSpec((_RPT, 128), lambda i, h: (i, h)),
        ],
        out_shape=[
            jax.ShapeDtypeStruct((N, H), jnp.float32),
            jax.ShapeDtypeStruct((N, H), jnp.float32),
        ],
    )(z1, z1, dinv, Wmu, bmu.reshape(1, H), Wlv, blv.reshape(1, H))

    return (mu, lv)


# speedup vs baseline: 9.8124x; 1.1303x over previous
import functools

import jax
import jax.numpy as jnp
from jax import lax
from jax.experimental import pallas as pl
from jax.experimental.pallas import tpu as pltpu
from jax.experimental.pallas import tpu_sc as plsc

_LANES = 128
_NC = 2
_NS = 16
_RPT = 640
_G = 16


def _deg_kernel(N2, R32):
    mesh = plsc.VectorSubcoreMesh(core_axis_name="c", subcore_axis_name="s",
                                  num_cores=_NC, num_subcores=_NS)

    @functools.partial(
        pl.kernel,
        mesh=mesh,
        out_type=jax.ShapeDtypeStruct((2 * N2, _LANES), jnp.float32),
        scratch_types=[
            pltpu.VMEM((R32, _LANES), jnp.int32),
            pltpu.VMEM((_LANES, _LANES), jnp.float32),
            pltpu.VMEM_SHARED((N2, _LANES), jnp.float32),
        ],
    )
    def body(dst_hbm, ones_hbm, out_hbm, dst_v, ones_v, acc_sh):
        c = lax.axis_index("c")
        s = lax.axis_index("s")
        wid = s * _NC + c
        pltpu.sync_copy(dst_hbm.at[wid], dst_v)
        pltpu.sync_copy(ones_hbm.at[pl.ds(0, _LANES)], ones_v)
        r0 = s * _RPT
        pltpu.sync_copy(ones_hbm.at[pl.ds(r0, _RPT)], acc_sh.at[pl.ds(r0, _RPT)])
        plsc.subcore_barrier()

        def step(j, carry):
            pltpu.sync_copy(ones_v, acc_sh.at[dst_v.at[j]], add=True)
            return carry

        lax.fori_loop(0, R32, step, 0)
        plsc.subcore_barrier()
        pltpu.sync_copy(acc_sh.at[pl.ds(r0, _RPT)],
                        out_hbm.at[pl.ds(c * N2 + r0, _RPT)])

    return body


def _agg_kernel(N2, R, C):
    mesh = plsc.VectorSubcoreMesh(core_axis_name="c", subcore_axis_name="s",
                                  num_cores=_NC, num_subcores=_NS)

    @functools.partial(
        pl.kernel,
        mesh=mesh,
        out_type=jax.ShapeDtypeStruct((2 * N2, _LANES), jnp.float32),
        scratch_types=[
            pltpu.VMEM((R, C), jnp.int32),
            pltpu.VMEM((_G, C), jnp.int32),
            pltpu.VMEM((C, _LANES), jnp.float32),
            pltpu.VMEM((C, _LANES), jnp.float32),
            pltpu.VMEM_SHARED((N2, _LANES), jnp.float32),
            pltpu.SemaphoreType.DMA,
            pltpu.SemaphoreType.DMA,
        ],
    )
    def body(y_hbm, src_hbm, dst_hbm, out_hbm, src_v, dst_st, rows0, rows1,
             acc_sh, sem0, sem1):
        c = lax.axis_index("c")
        s = lax.axis_index("s")
        pltpu.sync_copy(src_hbm.at[c, s], src_v)
        r0 = s * _RPT
        pltpu.sync_copy(y_hbm.at[pl.ds(c * N2 + r0, _RPT)],
                        acc_sh.at[pl.ds(r0, _RPT)])
        plsc.subcore_barrier()

        rows = (rows0, rows1)
        sems = (sem0, sem1)

        def group(g, carry):
            j0 = g * _G
            h = [pltpu.async_copy(y_hbm.at[src_v.at[j0]], rows0, sem0),
                 pltpu.async_copy(y_hbm.at[src_v.at[j0 + 1]], rows1, sem1)]
            pltpu.sync_copy(dst_hbm.at[s, g], dst_st)
            for k in range(_G):
                b = k % 2
                h[b].wait()
                pltpu.sync_copy(rows[b], acc_sh.at[dst_st.at[k]], add=True)
                if k + 2 < _G:
                    h[b] = pltpu.async_copy(y_hbm.at[src_v.at[j0 + k + 2]],
                                            rows[b], sems[b])
            return carry

        lax.fori_loop(0, R // _G, group, 0)
        plsc.subcore_barrier()
        pltpu.sync_copy(acc_sh.at[pl.ds(r0, _RPT)],
                        out_hbm.at[pl.ds(c * N2 + r0, _RPT)])

    return body


def _scale_body(d0_ref, d1_ref, x_ref, dinv_ref, y2_ref):
    dv = lax.rsqrt(d0_ref[:, 0:1] + d1_ref[:, 0:1] - 1.0)
    dinv_ref[...] = dv
    y2_ref[...] = x_ref[...] * dv


def _layer1_body(zl_ref, zr_ref, dv_ref, w_ref, b_ref, y_ref):
    dv = dv_ref[...]
    al = zl_ref[...] * dv
    ar = zr_ref[...] * dv
    pre = jnp.dot(al, w_ref[0:128, :], preferred_element_type=jnp.float32)
    pre = pre + jnp.dot(ar, w_ref[128:256, :], preferred_element_type=jnp.float32)
    pre = pre + b_ref[...]
    y_ref[...] = dv * jnp.maximum(pre, 0.0)


def _heads_body(zl_ref, zr_ref, dv_ref, wmu_ref, bmu_ref, wlv_ref, blv_ref,
                mu_ref, lv_ref):
    dv = dv_ref[...]
    al = zl_ref[...] * dv
    ar = zr_ref[...] * dv
    mu_ref[...] = (jnp.dot(al, wmu_ref[0:128, :], preferred_element_type=jnp.float32)
                   + jnp.dot(ar, wmu_ref[128:256, :], preferred_element_type=jnp.float32)
                   + bmu_ref[...])
    lv_ref[...] = (jnp.dot(al, wlv_ref[0:128, :], preferred_element_type=jnp.float32)
                   + jnp.dot(ar, wlv_ref[128:256, :], preferred_element_type=jnp.float32)
                   + blv_ref[...])


def kernel(x, edge_index, W1, b1, Wmu, bmu, Wlv, blv):
    N, D = x.shape
    E = edge_index.shape[1]
    H = W1.shape[1]
    N2 = _NS * _RPT
    nb = N2 // _RPT

    R32 = -(-E // (_NC * _NS * _LANES))
    Epad = _NC * _NS * R32 * _LANES
    C_AGG = _LANES
    R_AGG = Epad // (_NS * C_AGG)

    src = edge_index[0]
    dst = edge_index[1]
    padn = Epad - E
    src_p = jnp.concatenate([src, jnp.zeros((padn,), src.dtype)])
    dst_p = jnp.concatenate([dst, jnp.full((padn,), N, dst.dtype)])
    srcs = jnp.stack([src_p, src_p + N2]).reshape(_NC, _NS, R_AGG, C_AGG)
    dsts = dst_p.reshape(_NS, R_AGG // _G, _G, C_AGG)
    dst32 = dst_p.reshape(_NC * _NS, R32, _LANES)
    ones = jnp.ones((N2, _LANES), jnp.float32)

    degpart = _deg_kernel(N2, R32)(dst32, ones)

    dinv, y0 = pl.pallas_call(
        _scale_body,
        grid=(nb, 2),
        in_specs=[
            pl.BlockSpec((_RPT, 128), lambda i, h: (i, 0)),
            pl.BlockSpec((_RPT, 128), lambda i, h: (nb + i, 0)),
            pl.BlockSpec((_RPT, 128), lambda i, h: (i, h)),
        ],
        out_specs=[
            pl.BlockSpec((_RPT, 1), lambda i, h: (i, 0)),
            pl.BlockSpec((_RPT, 128), lambda i, h: (h * nb + i, 0)),
        ],
        out_shape=[
            jax.ShapeDtypeStruct((N, 1), jnp.float32),
            jax.ShapeDtypeStruct((2 * N2, 128), jnp.float32),
        ],
    )(degpart, degpart, x)

    agg = _agg_kernel(N2, R_AGG, C_AGG)
    z0 = agg(y0, srcs, dsts)

    y1 = pl.pallas_call(
        _layer1_body,
        grid=(nb, 2),
        in_specs=[
            pl.BlockSpec((_RPT, 128), lambda i, h: (i, 0)),
            pl.BlockSpec((_RPT, 128), lambda i, h: (nb + i, 0)),
            pl.BlockSpec((_RPT, 1), lambda i, h: (i, 0)),
            pl.BlockSpec((D, 128), lambda i, h: (0, h)),
            pl.BlockSpec((1, 128), lambda i, h: (0, h)),
        ],
        out_specs=pl.BlockSpec((_RPT, 128), lambda i, h: (h * nb + i, 0)),
        out_shape=jax.ShapeDtypeStruct((2 * N2, 128), jnp.float32),
    )(z0, z0, dinv, W1, b1.reshape(1, H))

    z1 = agg(y1, srcs, dsts)

    mu, lv = pl.pallas_call(
        _heads_body,
        grid=(nb, 2),
        in_specs=[
            pl.BlockSpec((_RPT, 128), lambda i, h: (i, 0)),
            pl.BlockSpec((_RPT, 128), lambda i, h: (nb + i, 0)),
            pl.BlockSpec((_RPT, 1), lambda i, h: (i, 0)),
            pl.BlockSpec((H, 128), lambda i, h: (0, h)),
            pl.BlockSpec((1, 128), lambda i, h: (0, h)),
            pl.BlockSpec((H, 128), lambda i, h: (0, h)),
            pl.BlockSpec((1, 128), lambda i, h: (0, h)),
        ],
        out_specs=[
            pl.BlockSpec((_RPT, 128), lambda i, h: (i, h)),
            pl.BlockSpec((_RPT, 128), lambda i, h: (i, h)),
        ],
        out_shape=[
            jax.ShapeDtypeStruct((N, H), jnp.float32),
            jax.ShapeDtypeStruct((N, H), jnp.float32),
        ],
    )(z1, z1, dinv, Wmu, bmu.reshape(1, H), Wlv, blv.reshape(1, H))

    return (mu, lv)
